# baseline (speedup 1.0000x reference)
import functools

import jax
import jax.numpy as jnp
from jax.experimental import pallas as pl
from jax.experimental.pallas import tpu as pltpu

N = 1_000_000
ELE_DIM = 32
MAT_FEAT = 83
TILE = 32_768


def _decoder_body(x_ref, w_ref, b_ref, out_ref, mask_ref):
    x = x_ref[...]
    mask = jnp.any(x != 0.0, axis=0)
    y = jnp.dot(w_ref[...], x, preferred_element_type=jnp.float32)
    y = y + b_ref[...]
    y = y * 0.5 * (1.0 + jax.lax.erf(y * 0.7071067811865476))
    s = 1.0 / (1.0 + jnp.exp2(y * -1.4426950408889634))
    maskf = (mask[None, :]).astype(jnp.float32)
    out_ref[...] = 0.5 + maskf * (s - 0.5)
    mask_ref[...] = mask


@functools.partial(jax.jit, static_argnames=("interpret",))
def _decoder(xt, w, b2, interpret=False):
    n = xt.shape[1]
    steps = pl.cdiv(n, TILE)
    out, mask = pl.pallas_call(
        _decoder_body,
        grid=(steps,),
        in_specs=[
            pl.BlockSpec((ELE_DIM, TILE), lambda i: (0, i)),
            pl.BlockSpec((MAT_FEAT, ELE_DIM), lambda i: (0, 0)),
            pl.BlockSpec((MAT_FEAT, 1), lambda i: (0, 0)),
        ],
        out_specs=[
            pl.BlockSpec((MAT_FEAT, TILE), lambda i: (0, i)),
            pl.BlockSpec((TILE,), lambda i: (i,)),
        ],
        out_shape=[
            jax.ShapeDtypeStruct((MAT_FEAT, n), jnp.float32),
            jax.ShapeDtypeStruct((steps * TILE,), jnp.bool_),
        ],
        interpret=interpret,
        compiler_params=None if interpret else pltpu.CompilerParams(
            dimension_semantics=("parallel",),
        ),
    )(xt, w, b2)
    return out, mask


def kernel(inputs, W, b):
    n = inputs.shape[0]
    b2 = b.reshape(MAT_FEAT, 1)
    out_t, mask = _decoder(inputs.T, W, b2)
    return out_t.T, mask[:n]
